# Initial kernel scaffold; baseline (speedup 1.0000x reference)
import jax
import jax.numpy as jnp
from jax.experimental import pallas as pl


def kernel(x, params):
    raise NotImplementedError("write your pallas kernel here")


# speedup vs baseline: 6.0676x; 6.0676x over previous
import functools

import jax
import jax.numpy as jnp
import numpy as np
from jax import lax
from jax.experimental import pallas as pl
from jax.experimental.pallas import tpu as pltpu
from jax.experimental.pallas import tpu_sc as plsc

_INT_MIN = np.int32(-(2 ** 31))


def _topk_body(q_ref, kt_ref, ia_ref, ib_ref, ic_ref, *, n, qb, m_per_chunk):
    b = pl.program_id(0)
    qi = pl.program_id(1)
    nchunk = n // 128
    q = q_ref[...]
    kt = kt_ref[0]
    xx = jnp.sum(kt * kt, axis=0, keepdims=True)
    s = 2.0 * jnp.dot(q, kt, preferred_element_type=jnp.float32) - xx
    bits = lax.bitcast_convert_type(s, jnp.int32)
    mono = bits ^ ((bits >> 31) & jnp.int32(0x7FFFFFFF))
    li = lax.broadcasted_iota(jnp.int32, (qb, n), 1) & 127
    skey = (mono & jnp.int32(-128)) | (127 - li)
    sk = skey.reshape(qb, nchunk, 128)
    cands = []
    for j in range(m_per_chunk):
        cm = jnp.max(sk, axis=2)
        cands.append(cm)
        if j < m_per_chunk - 1:
            sk = jnp.where(sk == cm[:, :, None], _INT_MIN, sk)
    cand = jnp.stack(cands, axis=2).reshape(qb, nchunk * m_per_chunk)
    posi = lax.broadcasted_iota(jnp.int32, (qb, nchunk * m_per_chunk), 1)
    base = b * n
    cols = []
    for _ in range(20):
        v = jnp.max(cand, axis=1, keepdims=True)
        eq = cand == v
        p = jnp.min(jnp.where(eq, posi, jnp.int32(1 << 30)), axis=1,
                    keepdims=True)
        cand = jnp.where(posi == p, _INT_MIN, cand)
        lane = 127 - (v & 127)
        m = ((p // m_per_chunk) << 7) | lane
        cols.append(m + base)
    rowi = lax.broadcasted_iota(jnp.int32, (qb, 1), 0)
    self_col = base + qi * qb + rowi
    ia_ref[...] = jnp.concatenate(cols + [self_col] * 4, axis=1)
    ib_ref[...] = jnp.concatenate(cols[:6] + [self_col] * 2, axis=1)
    ic_ref[...] = jnp.concatenate(cols[:8], axis=1)


def _topk_call(table, ktop, m_per_chunk=8, qb=256):
    bn, _ = table.shape
    bsz, _, n = ktop.shape
    nqb = n // qb
    grid = (bsz, nqb)
    body = functools.partial(_topk_body, n=n, qb=qb, m_per_chunk=m_per_chunk)
    return pl.pallas_call(
        body,
        grid=grid,
        in_specs=[
            pl.BlockSpec((qb, 16), lambda b, i: (b * nqb + i, 0)),
            pl.BlockSpec((1, 16, n), lambda b, i: (b, 0, 0)),
        ],
        out_specs=[
            pl.BlockSpec((qb, 24), lambda b, i: (b * nqb + i, 0)),
            pl.BlockSpec((qb, 8), lambda b, i: (b * nqb + i, 0)),
            pl.BlockSpec((qb, 8), lambda b, i: (b * nqb + i, 0)),
        ],
        out_shape=[
            jax.ShapeDtypeStruct((bn, 24), jnp.int32),
            jax.ShapeDtypeStruct((bn, 8), jnp.int32),
            jax.ShapeDtypeStruct((bn, 8), jnp.int32),
        ],
    )(table, ktop)


def _sc_gather(table, idx, chunk=2048):
    m = idx.shape[0]
    nw = 32
    per_w = m // nw
    nch = per_w // chunk
    mesh = plsc.VectorSubcoreMesh(core_axis_name="c", subcore_axis_name="s")

    @functools.partial(
        pl.kernel, mesh=mesh,
        compiler_params=pltpu.CompilerParams(use_tc_tiling_on_sc=False),
        out_type=jax.ShapeDtypeStruct((m, 16), jnp.float32),
        scratch_types=[
            pltpu.VMEM((chunk,), jnp.int32),
            pltpu.VMEM((chunk, 16), jnp.float32),
            pltpu.SemaphoreType.DMA,
        ],
    )
    def k(table_hbm, idx_hbm, out_hbm, idx_v, rows_v, sem):
        wid = lax.axis_index("s") * 2 + lax.axis_index("c")
        base = wid * per_w

        def body(i, carry):
            off = base + i * chunk
            pltpu.sync_copy(idx_hbm.at[pl.ds(off, chunk)], idx_v)
            pltpu.async_copy(table_hbm.at[idx_v], rows_v, sem).wait()
            pltpu.sync_copy(rows_v, out_hbm.at[pl.ds(off, chunk)])
            return carry

        lax.fori_loop(0, nch, body, 0)

    return k(table, idx)


def _gather(table, idx):
    return _sc_gather(table, idx)


def _edge_body(g_ref, x_ref, w1_ref, w2t_ref, w2_ref, b2_ref, w3_ref, b3_ref,
               o_ref, *, rep, rq):
    g = g_ref[...].reshape(rep * rq, 128)
    h1 = jnp.dot(g, w1_ref[...], preferred_element_type=jnp.float32)
    qt = jnp.dot(x_ref[...], w2t_ref[...], preferred_element_type=jnp.float32)
    if rep > 1:
        qt = jnp.concatenate([qt] * rep, axis=0)
    h1 = h1 + qt
    h1 = jnp.where(h1 >= 0, h1, 0.2 * h1)
    h2 = jnp.dot(h1, w2_ref[...], preferred_element_type=jnp.float32) + b2_ref[...]
    h2 = jnp.where(h2 >= 0, h2, 0.2 * h2)
    h3 = jnp.dot(h2, w3_ref[...], preferred_element_type=jnp.float32) + b3_ref[...]
    h3 = jnp.where(h3 >= 0, h3, 0.2 * h3)
    hm = h3[:, 0:64]
    for t in range(1, 8):
        hm = jnp.maximum(hm, h3[:, 64 * t:64 * (t + 1)])
    if rep > 1:
        hr = hm[0:rq]
        for t in range(1, rep):
            hr = jnp.maximum(hr, hm[rq * t:rq * (t + 1)])
        hm = hr
    o_ref[...] = hm


def _edge_call(gp, table, w1e, w2t, w2e, b2t, w3e, b3t, rq=256):
    rep, bn, _ = gp.shape
    nblk = bn // rq
    body = functools.partial(_edge_body, rep=rep, rq=rq)
    return pl.pallas_call(
        body,
        grid=(nblk,),
        in_specs=[
            pl.BlockSpec((rep, rq, 128), lambda i: (0, i, 0)),
            pl.BlockSpec((rq, 16), lambda i: (i, 0)),
            pl.BlockSpec(w1e.shape, lambda i: (0, 0)),
            pl.BlockSpec(w2t.shape, lambda i: (0, 0)),
            pl.BlockSpec(w2e.shape, lambda i: (0, 0)),
            pl.BlockSpec(b2t.shape, lambda i: (0, 0)),
            pl.BlockSpec(w3e.shape, lambda i: (0, 0)),
            pl.BlockSpec(b3t.shape, lambda i: (0, 0)),
        ],
        out_specs=pl.BlockSpec((rq, 64), lambda i: (i, 0)),
        out_shape=jax.ShapeDtypeStruct((bn, 64), jnp.float32),
    )(gp, table, w1e, w2t, w2e, b2t, w3e, b3t)


def _d1_body(x_ref, w6_ref, b6_ref, o_ref, *, nblk):
    i = pl.program_id(1)
    h = jnp.dot(x_ref[...], w6_ref[...], preferred_element_type=jnp.float32)
    h = h + b6_ref[...]
    h = jnp.where(h >= 0, h, 0.2 * h)
    bm = jnp.max(h, axis=0, keepdims=True)[None]
    @pl.when(i == 0)
    def _():
        o_ref[...] = bm
    @pl.when(i > 0)
    def _():
        o_ref[...] = jnp.maximum(o_ref[...], bm)


def _d1_call(x123, w6t, b6, bsz, rq=512):
    bn, _ = x123.shape
    n = bn // bsz
    nblk = n // rq
    body = functools.partial(_d1_body, nblk=nblk)
    return pl.pallas_call(
        body,
        grid=(bsz, nblk),
        in_specs=[
            pl.BlockSpec((rq, 192), lambda b, i: (b * nblk + i, 0)),
            pl.BlockSpec(w6t.shape, lambda b, i: (0, 0)),
            pl.BlockSpec(b6.shape, lambda b, i: (0, 0)),
        ],
        out_specs=pl.BlockSpec((1, 1, 1024), lambda b, i: (b, 0, 0)),
        out_shape=jax.ShapeDtypeStruct((bsz, 1, 1024), jnp.float32),
    )(x123, w6t, b6)


def _d2_body(x_ref, m_ref, w7a_ref, w7b_ref, b7_ref, w8_ref, b8_ref, w9_ref,
             o_ref):
    m6 = m_ref[0]
    t7 = jnp.dot(m6, w7a_ref[...], preferred_element_type=jnp.float32)
    t7 = t7 + b7_ref[...]
    h7 = jnp.dot(x_ref[...], w7b_ref[...], preferred_element_type=jnp.float32)
    h7 = h7 + t7
    h7 = jnp.where(h7 >= 0, h7, 0.2 * h7)
    h8 = jnp.dot(h7, w8_ref[...], preferred_element_type=jnp.float32) + b8_ref[...]
    h8 = jnp.where(h8 >= 0, h8, 0.2 * h8)
    h9 = jnp.dot(h8, w9_ref[...], preferred_element_type=jnp.float32)
    o_ref[...] = jnp.transpose(h9)[None, 0:13, :]


def _d2_call(x123, m6, w7a, w7b, b7, w8t, b8, w9t, bsz, rq=512):
    bn, _ = x123.shape
    n = bn // bsz
    nblk = n // rq
    return pl.pallas_call(
        _d2_body,
        grid=(bsz, nblk),
        in_specs=[
            pl.BlockSpec((rq, 192), lambda b, i: (b * nblk + i, 0)),
            pl.BlockSpec((1, 1, 1024), lambda b, i: (b, 0, 0)),
            pl.BlockSpec(w7a.shape, lambda b, i: (0, 0)),
            pl.BlockSpec(w7b.shape, lambda b, i: (0, 0)),
            pl.BlockSpec(b7.shape, lambda b, i: (0, 0)),
            pl.BlockSpec(w8t.shape, lambda b, i: (0, 0)),
            pl.BlockSpec(b8.shape, lambda b, i: (0, 0)),
            pl.BlockSpec(w9t.shape, lambda b, i: (0, 0)),
        ],
        out_specs=pl.BlockSpec((1, 13, rq), lambda b, i: (b, 0, i)),
        out_shape=jax.ShapeDtypeStruct((bsz, 13, n), jnp.float32),
    )(x123, m6, w7a, w7b, b7, w8t, b8, w9t)


def _fold_branch(wa, wb, wc, sa, ba, sb, bb, sc_, bc):
    wa9 = wa[:, :9]
    wd = wa[:, 9:] - wa9
    eye8 = jnp.eye(8, dtype=jnp.float32)
    w1p = jnp.zeros((16, 32), jnp.float32).at[:9].set(wa9.T * sa[None, :])
    w1e = jnp.kron(eye8, w1p)
    w2q = (jnp.zeros((16, 32), jnp.float32)
           .at[:9].set(wd.T * sa[None, :])
           .at[15].set(ba))
    w2t = jnp.tile(w2q, (1, 8))
    w2e = jnp.kron(eye8, wb.T * sb[None, :])
    b2t = jnp.tile(bb, 8)[None, :]
    w3e = jnp.kron(eye8, wc.T * sc_[None, :])
    b3t = jnp.tile(bc, 8)[None, :]
    return w1e, w2t, w2e, b2t, w3e, b3t


def _bn_scale(p, name):
    return p[name + '_g'] / jnp.sqrt(1.0 + 1e-5), p[name + '_b']


def kernel(x, params):
    bsz, c, n = x.shape
    bn = bsz * n
    xt = jnp.transpose(x, (0, 2, 1)).reshape(bn, c)
    table = jnp.concatenate(
        [xt, jnp.zeros((bn, 6), jnp.float32), jnp.ones((bn, 1), jnp.float32)],
        axis=1)
    ktop = jnp.concatenate([x, jnp.zeros((bsz, 7, n), jnp.float32)], axis=1)

    ia, ib, ic = _topk_call(table, ktop)

    idx1 = ia.reshape(bn, 3, 8).transpose(1, 0, 2).reshape(-1)
    idx_all = jnp.concatenate([idx1, ib.reshape(-1), ic.reshape(-1)])
    g = _gather(table, idx_all)
    m1 = bn * 24
    m2 = bn * 8
    g1 = g[:m1].reshape(3, bn, 128)
    g2 = g[m1:m1 + m2].reshape(1, bn, 128)
    g3 = g[m1 + m2:].reshape(1, bn, 128)

    p = params
    f1 = _fold_branch(p['W1'], p['W1_1'], p['W2'],
                      *_bn_scale(p, 'bn1'), *_bn_scale(p, 'bn1_1'),
                      *_bn_scale(p, 'bn2'))
    f2 = _fold_branch(p['W3'], p['W3_1'], p['W4'],
                      *_bn_scale(p, 'bn3'), *_bn_scale(p, 'bn3_1'),
                      *_bn_scale(p, 'bn4'))
    f3 = _fold_branch(p['W5'], p['W5_1'], p['W5_2'],
                      *_bn_scale(p, 'bn5'), *_bn_scale(p, 'bn5_1'),
                      *_bn_scale(p, 'bn5_2'))
    x1 = _edge_call(g1, table, *f1)
    x2 = _edge_call(g2, table, *f2)
    x3 = _edge_call(g3, table, *f3)
    x123 = jnp.concatenate([x1, x2, x3], axis=1)

    s6, b6 = _bn_scale(p, 'bn6')
    w6t = p['W6'].T * s6[None, :]
    m6 = _d1_call(x123, w6t, b6[None, :], bsz)

    s7, b7 = _bn_scale(p, 'bn7')
    w7a = p['W7'][:, :1024].T * s7[None, :]
    w7b = p['W7'][:, 1024:].T * s7[None, :]
    s8, b8 = _bn_scale(p, 'bn8')
    w8t = p['W8'].T * s8[None, :]
    w9t = jnp.zeros((256, 16), jnp.float32).at[:, :13].set(p['W9'].T)
    return _d2_call(x123, m6, w7a, w7b, b7[None, :], w8t, b8[None, :], w9t,
                    bsz)
